# Initial kernel scaffold; baseline (speedup 1.0000x reference)
import jax
import jax.numpy as jnp
from jax.experimental import pallas as pl


def kernel(user_ids, item_ids, inter_u, inter_i, user_emb, item_emb, layer_weights):
    raise NotImplementedError("write your pallas kernel here")


# speedup vs baseline: 12.7959x; 12.7959x over previous
import jax
import jax.numpy as jnp
from jax import lax
from jax.experimental import pallas as pl
from jax.experimental.pallas import tpu as pltpu
from jax.experimental.pallas import tpu_sc as plsc

NU = 25000
D = 64
B = 16384

NC, NS, L = 2, 16, 16
NPAD = 25088
RPT = NPAD // NS
RCH = 112
NRCH = RPT // RCH
ZROWS = NPAD + 8
EPAD = 401408
EPT = EPAD // NS
ECH = 128
NECH = EPT // ECH
BPT = B // (NC * NS)
BCH = 128

_MESH = plsc.VectorSubcoreMesh(
    core_axis_name="c", subcore_axis_name="s", num_cores=NC, num_subcores=NS)

_f32 = jnp.float32
_i32 = jnp.int32


def _fori(n, body):
    lax.fori_loop(0, n, lambda i, c: (body(i), c)[1], 0)


def _fill_zero_2d(ref, rows):
    zero = jnp.zeros((L,), _f32)

    def row(r):
        for j in range(D // L):
            ref[r, pl.ds(j * L, L)] = zero
    _fori(rows, row)


def _rsqrt16(x):
    i = lax.bitcast_convert_type(x, _i32)
    y = lax.bitcast_convert_type(jnp.int32(0x5F3759DF) - (i >> 1), _f32)
    for _ in range(3):
        y = y * (1.5 - 0.5 * x * y * y)
    return jnp.where(x >= 0.5, y, 0.0)


def _prologue_body(iu, ii, ue, ie, lw0v,
                   z_u, z_i, f_u, f_i, dis_u, dis_i,
                   deg_sh, idx_v, ones_v, col_v, dis_v, row_v, zv, fv, lw_v):
    c = lax.axis_index("c")
    s = lax.axis_index("s")
    base = s * RPT

    def ones_row(r):
        ones_v[r] = jnp.ones((L,), _f32)
    _fori(ECH, ones_row)

    def zcol(r):
        col_v[r] = jnp.zeros((L,), _f32)
    _fori(RPT, zcol)
    pltpu.sync_copy(lw0v, lw_v)

    def half(inter_ref, emb_ref, z_out, f_out, dis_out):
        pltpu.sync_copy(col_v, deg_sh.at[pl.ds(base, RPT)])

        @pl.when(s == NS - 1)
        def _():
            pltpu.sync_copy(col_v.at[pl.ds(0, 8)], deg_sh.at[pl.ds(NPAD, 8)])
        plsc.subcore_barrier()

        def echunk(ci):
            off = s * EPT + ci * ECH
            pltpu.sync_copy(inter_ref.at[pl.ds(off, ECH)], idx_v)
            pltpu.sync_copy(ones_v, deg_sh.at[idx_v], add=True)
        _fori(NECH, echunk)
        plsc.subcore_barrier()

        pltpu.sync_copy(deg_sh.at[pl.ds(base, RPT)], col_v)

        def drow(r):
            dis_v[r] = _rsqrt16(col_v[r])
        _fori(RPT, drow)
        pltpu.sync_copy(dis_v, dis_out.at[pl.ds(base, RPT)])

        lw0 = lw_v[...]

        def rchunk(rc):
            rbase = base + rc * RCH
            pltpu.sync_copy(emb_ref.at[pl.ds(rbase, RCH)], row_v)

            def row(r):
                d = dis_v[rc * RCH + r]
                for j in range(D // L):
                    x = row_v[r, pl.ds(j * L, L)]
                    zv[r, pl.ds(j * L, L)] = d * x
                    fv[r, pl.ds(j * L, L)] = lw0 * x
            _fori(RCH, row)
            pltpu.sync_copy(zv, z_out.at[pl.ds(rbase, RCH)])
            pltpu.sync_copy(fv, f_out.at[pl.ds(rbase, RCH)])
        _fori(NRCH, rchunk)

        @pl.when(s == NS - 1)
        def _():
            _fill_zero_2d(zv, 8)
            pltpu.sync_copy(zv.at[pl.ds(0, 8)], z_out.at[pl.ds(NPAD, 8)])

    @pl.when(c == 0)
    def _():
        half(iu, ue, z_u, f_u, dis_u)

    @pl.when(c == 1)
    def _():
        half(ii, ie, z_i, f_i, dis_i)


def _layer_body(iu, ii, z_u, z_i, dis_u, dis_i, f_u, f_i, lwlv,
                z_u2, z_i2, f_u2, f_i2,
                acc_sh, idx_g, idx_s, rows_v, acc_v, fin_v, dis_c, lw_v):
    c = lax.axis_index("c")
    s = lax.axis_index("s")
    base = s * RPT

    pltpu.sync_copy(lwlv, lw_v)

    def half(gidx_ref, sidx_ref, z_tab, dis_ref, f_in, z_out, f_out):
        _fill_zero_2d(fin_v, RCH)

        def zchunk(rc):
            pltpu.sync_copy(fin_v, acc_sh.at[pl.ds(base + rc * RCH, RCH)])
        _fori(NRCH, zchunk)

        @pl.when(s == NS - 1)
        def _():
            pltpu.sync_copy(fin_v.at[pl.ds(0, 8)], acc_sh.at[pl.ds(NPAD, 8)])
        plsc.subcore_barrier()

        def echunk(ci):
            off = s * EPT + ci * ECH
            pltpu.sync_copy(gidx_ref.at[pl.ds(off, ECH)], idx_g)
            pltpu.sync_copy(sidx_ref.at[pl.ds(off, ECH)], idx_s)
            pltpu.sync_copy(z_tab.at[idx_g], rows_v)
            pltpu.sync_copy(rows_v, acc_sh.at[idx_s], add=True)
        _fori(NECH, echunk)
        plsc.subcore_barrier()

        lwl = lw_v[...]

        def rchunk(rc):
            rbase = base + rc * RCH
            pltpu.sync_copy(acc_sh.at[pl.ds(rbase, RCH)], acc_v)
            pltpu.sync_copy(f_in.at[pl.ds(rbase, RCH)], fin_v)
            pltpu.sync_copy(dis_ref.at[pl.ds(rbase, RCH)], dis_c)

            def row(r):
                d = dis_c[r]
                for j in range(D // L):
                    sl = pl.ds(j * L, L)
                    t = d * acc_v[r, sl]
                    fin_v[r, sl] = fin_v[r, sl] + lwl * t
                    acc_v[r, sl] = d * t
            _fori(RCH, row)
            pltpu.sync_copy(acc_v, z_out.at[pl.ds(rbase, RCH)])
            pltpu.sync_copy(fin_v, f_out.at[pl.ds(rbase, RCH)])
        _fori(NRCH, rchunk)

        @pl.when(s == NS - 1)
        def _():
            _fill_zero_2d(fin_v, 8)
            pltpu.sync_copy(fin_v.at[pl.ds(0, 8)], z_out.at[pl.ds(NPAD, 8)])

    @pl.when(c == 0)
    def _():
        half(ii, iu, z_i, dis_u, f_u, z_u2, f_u2)

    @pl.when(c == 1)
    def _():
        half(iu, ii, z_u, dis_i, f_i, z_i2, f_i2)


def _score_body(uid, iid, f_u, f_i, out,
                uid_v, iid_v, urows, irows, sc_v):
    c = lax.axis_index("c")
    s = lax.axis_index("s")
    wid = s * NC + c
    tb = wid * BPT

    def bchunk(bc):
        off = tb + bc * BCH
        pltpu.sync_copy(uid.at[pl.ds(off, BCH)], uid_v)
        pltpu.sync_copy(iid.at[pl.ds(off, BCH)], iid_v)
        pltpu.sync_copy(f_u.at[uid_v], urows)
        pltpu.sync_copy(f_i.at[iid_v], irows)

        lanes = lax.iota(_i32, L)

        def grp(g):
            def rb(k, acc):
                r = g * L + k
                p = jnp.zeros((L,), _f32)
                for j in range(D // L):
                    sl = pl.ds(j * L, L)
                    p = p + urows[r, sl] * irows[r, sl]
                for sh in (1, 2, 4, 8):
                    p = p + jnp.take(p, lanes ^ sh)
                return jnp.where(lanes == k, p, acc)
            acc = lax.fori_loop(0, L, rb, jnp.zeros((L,), _f32))
            sc_v[pl.ds(g * L, L)] = acc
        _fori(BCH // L, grp)
        pltpu.sync_copy(sc_v, out.at[pl.ds(off, BCH)])
    _fori(BPT // BCH, bchunk)


def _node_struct():
    return jax.ShapeDtypeStruct((NPAD, D), _f32)


def _mk_prologue():
    return pl.kernel(
        _prologue_body,
        out_type=[
            jax.ShapeDtypeStruct((ZROWS, D), _f32),
            jax.ShapeDtypeStruct((ZROWS, D), _f32),
            _node_struct(),
            _node_struct(),
            jax.ShapeDtypeStruct((NPAD, L), _f32),
            jax.ShapeDtypeStruct((NPAD, L), _f32),
        ],
        mesh=_MESH,
        compiler_params=pltpu.CompilerParams(use_tc_tiling_on_sc=False),
        scratch_types=[
            pltpu.VMEM_SHARED((ZROWS, L), _f32),
            pltpu.VMEM((ECH,), _i32),
            pltpu.VMEM((ECH, L), _f32),
            pltpu.VMEM((RPT, L), _f32),
            pltpu.VMEM((RPT, L), _f32),
            pltpu.VMEM((RCH, D), _f32),
            pltpu.VMEM((RCH, D), _f32),
            pltpu.VMEM((RCH, D), _f32),
            pltpu.VMEM((L,), _f32),
        ],
    )


def _mk_layer():
    return pl.kernel(
        _layer_body,
        out_type=[
            jax.ShapeDtypeStruct((ZROWS, D), _f32),
            jax.ShapeDtypeStruct((ZROWS, D), _f32),
            _node_struct(),
            _node_struct(),
        ],
        mesh=_MESH,
        compiler_params=pltpu.CompilerParams(use_tc_tiling_on_sc=False),
        scratch_types=[
            pltpu.VMEM_SHARED((ZROWS, D), _f32),
            pltpu.VMEM((ECH,), _i32),
            pltpu.VMEM((ECH,), _i32),
            pltpu.VMEM((ECH, D), _f32),
            pltpu.VMEM((RCH, D), _f32),
            pltpu.VMEM((RCH, D), _f32),
            pltpu.VMEM((RCH, L), _f32),
            pltpu.VMEM((L,), _f32),
        ],
    )


def _mk_score():
    return pl.kernel(
        _score_body,
        out_type=jax.ShapeDtypeStruct((B,), _f32),
        mesh=_MESH,
        compiler_params=pltpu.CompilerParams(use_tc_tiling_on_sc=False),
        scratch_types=[
            pltpu.VMEM((BCH,), _i32),
            pltpu.VMEM((BCH,), _i32),
            pltpu.VMEM((BCH, D), _f32),
            pltpu.VMEM((BCH, D), _f32),
            pltpu.VMEM((BCH,), _f32),
        ],
    )


def kernel(user_ids, item_ids, inter_u, inter_i, user_emb, item_emb,
           layer_weights):
    lw = jax.nn.softmax(layer_weights)
    lw_splats = [jnp.full((L,), lw[k], _f32) for k in range(4)]
    ue_p = jnp.pad(user_emb, ((0, NPAD - NU), (0, 0)))
    ie_p = jnp.pad(item_emb, ((0, NPAD - NU), (0, 0)))
    iu_p = jnp.pad(inter_u, (0, EPAD - inter_u.shape[0]), constant_values=NPAD)
    ii_p = jnp.pad(inter_i, (0, EPAD - inter_i.shape[0]), constant_values=NPAD)

    z_u, z_i, f_u, f_i, dis_u, dis_i = _mk_prologue()(
        iu_p, ii_p, ue_p, ie_p, lw_splats[0])
    layer = _mk_layer()
    for l in range(1, 4):
        z_u, z_i, f_u, f_i = layer(
            iu_p, ii_p, z_u, z_i, dis_u, dis_i, f_u, f_i, lw_splats[l])
    return _mk_score()(user_ids, item_ids, f_u, f_i)


# speedup vs baseline: 20.2892x; 1.5856x over previous
import jax
import jax.numpy as jnp
from jax import lax
from jax.experimental import pallas as pl
from jax.experimental.pallas import tpu as pltpu
from jax.experimental.pallas import tpu_sc as plsc

NU = 25000
D = 64
B = 16384

NC, NS, L = 2, 16, 16
NPAD = 25088
RPT = NPAD // NS
RCH = 112
NRCH = RPT // RCH
ZROWS = NPAD + 8
EPAD = 401408
EPT = EPAD // NS
ECH = 128
NECH = EPT // ECH
BPT = B // (NC * NS)
BCH = 128

ECH2 = 64
NB = 4
QE = NB * ECH2
NQ = EPT // QE
RCHL = 56
NRCHL = RPT // RCHL

_MESH = plsc.VectorSubcoreMesh(
    core_axis_name="c", subcore_axis_name="s", num_cores=NC, num_subcores=NS)

_f32 = jnp.float32
_i32 = jnp.int32


def _fori(n, body):
    lax.fori_loop(0, n, lambda i, c: (body(i), c)[1], 0)


def _fill_zero_2d(ref, rows):
    zero = jnp.zeros((L,), _f32)

    def row(r):
        for j in range(D // L):
            ref[r, pl.ds(j * L, L)] = zero
    _fori(rows, row)


def _rsqrt16(x):
    i = lax.bitcast_convert_type(x, _i32)
    y = lax.bitcast_convert_type(jnp.int32(0x5F3759DF) - (i >> 1), _f32)
    for _ in range(3):
        y = y * (1.5 - 0.5 * x * y * y)
    return jnp.where(x >= 0.5, y, 0.0)


def _prologue_body(iu, ii, ue, ie, lw0v,
                   z_u, z_i, f_u, f_i, dis_u, dis_i,
                   deg_sh, idx_v, ones_v, col_v, dis_v, row_v, zv, fv, lw_v):
    c = lax.axis_index("c")
    s = lax.axis_index("s")
    base = s * RPT

    def ones_row(r):
        ones_v[r] = jnp.ones((L,), _f32)
    _fori(ECH, ones_row)

    def zcol(r):
        col_v[r] = jnp.zeros((L,), _f32)
    _fori(RPT, zcol)
    pltpu.sync_copy(lw0v, lw_v)

    def half(inter_ref, emb_ref, z_out, f_out, dis_out):
        pltpu.sync_copy(col_v, deg_sh.at[pl.ds(base, RPT)])

        @pl.when(s == NS - 1)
        def _():
            pltpu.sync_copy(col_v.at[pl.ds(0, 8)], deg_sh.at[pl.ds(NPAD, 8)])
        plsc.subcore_barrier()

        def echunk(ci):
            off = s * EPT + ci * ECH
            pltpu.sync_copy(inter_ref.at[pl.ds(off, ECH)], idx_v)
            pltpu.sync_copy(ones_v, deg_sh.at[idx_v], add=True)
        _fori(NECH, echunk)
        plsc.subcore_barrier()

        pltpu.sync_copy(deg_sh.at[pl.ds(base, RPT)], col_v)

        def drow(r):
            dis_v[r] = _rsqrt16(col_v[r])
        _fori(RPT, drow)
        pltpu.sync_copy(dis_v, dis_out.at[pl.ds(base, RPT)])

        lw0 = lw_v[...]

        def rchunk(rc):
            rbase = base + rc * RCH
            pltpu.sync_copy(emb_ref.at[pl.ds(rbase, RCH)], row_v)

            def row(r):
                d = dis_v[rc * RCH + r]
                for j in range(D // L):
                    x = row_v[r, pl.ds(j * L, L)]
                    zv[r, pl.ds(j * L, L)] = d * x
                    fv[r, pl.ds(j * L, L)] = lw0 * x
            _fori(RCH, row)
            pltpu.sync_copy(zv, z_out.at[pl.ds(rbase, RCH)])
            pltpu.sync_copy(fv, f_out.at[pl.ds(rbase, RCH)])
        _fori(NRCH, rchunk)

        @pl.when(s == NS - 1)
        def _():
            _fill_zero_2d(zv, 8)
            pltpu.sync_copy(zv.at[pl.ds(0, 8)], z_out.at[pl.ds(NPAD, 8)])

    @pl.when(c == 0)
    def _():
        half(iu, ue, z_u, f_u, dis_u)

    @pl.when(c == 1)
    def _():
        half(ii, ie, z_i, f_i, dis_i)


def _layer_body(iu, ii, z_u, z_i, dis_u, dis_i, f_u, f_i, lwlv,
                z_u2, z_i2, f_u2, f_i2,
                acc_sh, idxg_a, idxg_b, idxs_a, idxs_b,
                rows0, rows1, rows2, rows3, acc_v, fin_v, dis_c, lw_v,
                sg0, sg1, sg2, sg3, ss0, ss1, ss2, ss3, sia, sib):
    c = lax.axis_index("c")
    s = lax.axis_index("s")
    base = s * RPT
    rows = [rows0, rows1, rows2, rows3]
    sem_g = [sg0, sg1, sg2, sg3]
    sem_s = [ss0, ss1, ss2, ss3]
    idx_g = [idxg_a, idxg_b]
    idx_s = [idxs_a, idxs_b]
    sem_i = [sia, sib]

    pltpu.sync_copy(lwlv, lw_v)

    def half(gidx_ref, sidx_ref, z_tab, dis_ref, f_in, z_out, f_out):
        _fill_zero_2d(fin_v, RCHL)

        def zchunk(rc):
            pltpu.sync_copy(fin_v, acc_sh.at[pl.ds(base + rc * RCHL, RCHL)])
        _fori(NRCHL, zchunk)

        @pl.when(s == NS - 1)
        def _():
            pltpu.sync_copy(fin_v.at[pl.ds(0, 8)], acc_sh.at[pl.ds(NPAD, 8)])
        plsc.subcore_barrier()

        qrow = s * (EPT // ECH2)

        def idx_issue(q, p):
            r0 = qrow + q * NB
            pltpu.async_copy(gidx_ref.at[pl.ds(r0, NB)], idx_g[p], sem_i[p])
            pltpu.async_copy(sidx_ref.at[pl.ds(r0, NB)], idx_s[p], sem_i[p])

        def idx_wait(p):
            pltpu.make_async_copy(
                gidx_ref.at[pl.ds(0, NB)], idx_g[p], sem_i[p]).wait()
            pltpu.make_async_copy(
                gidx_ref.at[pl.ds(0, NB)], idx_s[p], sem_i[p]).wait()

        def scat_wait(p):
            for k in range(NB):
                pltpu.make_async_copy(
                    rows[k], acc_sh.at[idx_s[p].at[k]], sem_s[k]).wait()

        def quad_step(q, p, wait_prev, prefetch):
            if wait_prev:
                scat_wait(1 - p)
            if prefetch is None:
                idx_issue(q + 1, 1 - p)
            else:
                pl.when(prefetch)(lambda: idx_issue(q + 1, 1 - p))
            idx_wait(p)
            for k in range(NB):
                pltpu.async_copy(z_tab.at[idx_g[p].at[k]], rows[k], sem_g[k])
            for k in range(NB):
                pltpu.make_async_copy(
                    z_tab.at[idx_g[p].at[k]], rows[k], sem_g[k]).wait()
                pltpu.async_copy(
                    rows[k], acc_sh.at[idx_s[p].at[k]], sem_s[k], add=True)

        idx_issue(0, 0)
        quad_step(0, 0, False, None)
        quad_step(1, 1, True, None)

        def qpair(t):
            quad_step(2 * t, 0, True, None)
            quad_step(2 * t + 1, 1, True, t < NQ // 2 - 1)
        lax.fori_loop(1, NQ // 2, lambda t, cc: (qpair(t), cc)[1], 0)
        scat_wait(1)
        plsc.subcore_barrier()

        lwl = lw_v[...]

        def rchunk(rc):
            rbase = base + rc * RCHL
            pltpu.sync_copy(acc_sh.at[pl.ds(rbase, RCHL)], acc_v)
            pltpu.sync_copy(f_in.at[pl.ds(rbase, RCHL)], fin_v)
            pltpu.sync_copy(dis_ref.at[pl.ds(rbase, RCHL)], dis_c)

            def row(r):
                d = dis_c[r]
                for j in range(D // L):
                    sl = pl.ds(j * L, L)
                    t = d * acc_v[r, sl]
                    fin_v[r, sl] = fin_v[r, sl] + lwl * t
                    acc_v[r, sl] = d * t
            _fori(RCHL, row)
            pltpu.sync_copy(acc_v, z_out.at[pl.ds(rbase, RCHL)])
            pltpu.sync_copy(fin_v, f_out.at[pl.ds(rbase, RCHL)])
        _fori(NRCHL, rchunk)

        @pl.when(s == NS - 1)
        def _():
            _fill_zero_2d(fin_v, 8)
            pltpu.sync_copy(fin_v.at[pl.ds(0, 8)], z_out.at[pl.ds(NPAD, 8)])

    @pl.when(c == 0)
    def _():
        half(ii, iu, z_i, dis_u, f_u, z_u2, f_u2)

    @pl.when(c == 1)
    def _():
        half(iu, ii, z_u, dis_i, f_i, z_i2, f_i2)


def _score_body(uid, iid, f_u, f_i, out,
                uid_v, iid_v, urows, irows, sc_v):
    c = lax.axis_index("c")
    s = lax.axis_index("s")
    wid = s * NC + c
    tb = wid * BPT

    def bchunk(bc):
        off = tb + bc * BCH
        pltpu.sync_copy(uid.at[pl.ds(off, BCH)], uid_v)
        pltpu.sync_copy(iid.at[pl.ds(off, BCH)], iid_v)
        pltpu.sync_copy(f_u.at[uid_v], urows)
        pltpu.sync_copy(f_i.at[iid_v], irows)

        lanes = lax.iota(_i32, L)

        def grp(g):
            def rb(k, acc):
                r = g * L + k
                p = jnp.zeros((L,), _f32)
                for j in range(D // L):
                    sl = pl.ds(j * L, L)
                    p = p + urows[r, sl] * irows[r, sl]
                for sh in (1, 2, 4, 8):
                    p = p + jnp.take(p, lanes ^ sh)
                return jnp.where(lanes == k, p, acc)
            acc = lax.fori_loop(0, L, rb, jnp.zeros((L,), _f32))
            sc_v[pl.ds(g * L, L)] = acc
        _fori(BCH // L, grp)
        pltpu.sync_copy(sc_v, out.at[pl.ds(off, BCH)])
    _fori(BPT // BCH, bchunk)


def _node_struct():
    return jax.ShapeDtypeStruct((NPAD, D), _f32)


def _mk_prologue():
    return pl.kernel(
        _prologue_body,
        out_type=[
            jax.ShapeDtypeStruct((ZROWS, D), _f32),
            jax.ShapeDtypeStruct((ZROWS, D), _f32),
            _node_struct(),
            _node_struct(),
            jax.ShapeDtypeStruct((NPAD, L), _f32),
            jax.ShapeDtypeStruct((NPAD, L), _f32),
        ],
        mesh=_MESH,
        compiler_params=pltpu.CompilerParams(use_tc_tiling_on_sc=False),
        scratch_types=[
            pltpu.VMEM_SHARED((ZROWS, L), _f32),
            pltpu.VMEM((ECH,), _i32),
            pltpu.VMEM((ECH, L), _f32),
            pltpu.VMEM((RPT, L), _f32),
            pltpu.VMEM((RPT, L), _f32),
            pltpu.VMEM((RCH, D), _f32),
            pltpu.VMEM((RCH, D), _f32),
            pltpu.VMEM((RCH, D), _f32),
            pltpu.VMEM((L,), _f32),
        ],
    )


def _mk_layer():
    return pl.kernel(
        _layer_body,
        out_type=[
            jax.ShapeDtypeStruct((ZROWS, D), _f32),
            jax.ShapeDtypeStruct((ZROWS, D), _f32),
            _node_struct(),
            _node_struct(),
        ],
        mesh=_MESH,
        compiler_params=pltpu.CompilerParams(use_tc_tiling_on_sc=False),
        scratch_types=[
            pltpu.VMEM_SHARED((ZROWS, D), _f32),
            pltpu.VMEM((NB, ECH2), _i32),
            pltpu.VMEM((NB, ECH2), _i32),
            pltpu.VMEM((NB, ECH2), _i32),
            pltpu.VMEM((NB, ECH2), _i32),
            pltpu.VMEM((ECH2, D), _f32),
            pltpu.VMEM((ECH2, D), _f32),
            pltpu.VMEM((ECH2, D), _f32),
            pltpu.VMEM((ECH2, D), _f32),
            pltpu.VMEM((RCHL, D), _f32),
            pltpu.VMEM((RCHL, D), _f32),
            pltpu.VMEM((RCHL, L), _f32),
            pltpu.VMEM((L,), _f32),
            pltpu.SemaphoreType.DMA,
            pltpu.SemaphoreType.DMA,
            pltpu.SemaphoreType.DMA,
            pltpu.SemaphoreType.DMA,
            pltpu.SemaphoreType.DMA,
            pltpu.SemaphoreType.DMA,
            pltpu.SemaphoreType.DMA,
            pltpu.SemaphoreType.DMA,
            pltpu.SemaphoreType.DMA,
            pltpu.SemaphoreType.DMA,
        ],
    )


def _mk_score():
    return pl.kernel(
        _score_body,
        out_type=jax.ShapeDtypeStruct((B,), _f32),
        mesh=_MESH,
        compiler_params=pltpu.CompilerParams(use_tc_tiling_on_sc=False),
        scratch_types=[
            pltpu.VMEM((BCH,), _i32),
            pltpu.VMEM((BCH,), _i32),
            pltpu.VMEM((BCH, D), _f32),
            pltpu.VMEM((BCH, D), _f32),
            pltpu.VMEM((BCH,), _f32),
        ],
    )


def kernel(user_ids, item_ids, inter_u, inter_i, user_emb, item_emb,
           layer_weights):
    lw = jax.nn.softmax(layer_weights)
    lw_splats = [jnp.full((L,), lw[k], _f32) for k in range(4)]
    ue_p = jnp.pad(user_emb, ((0, NPAD - NU), (0, 0)))
    ie_p = jnp.pad(item_emb, ((0, NPAD - NU), (0, 0)))
    iu_p = jnp.pad(inter_u, (0, EPAD - inter_u.shape[0]), constant_values=NPAD)
    ii_p = jnp.pad(inter_i, (0, EPAD - inter_i.shape[0]), constant_values=NPAD)

    iu2 = iu_p.reshape(EPAD // ECH2, ECH2)
    ii2 = ii_p.reshape(EPAD // ECH2, ECH2)

    z_u, z_i, f_u, f_i, dis_u, dis_i = _mk_prologue()(
        iu_p, ii_p, ue_p, ie_p, lw_splats[0])
    layer = _mk_layer()
    for l in range(1, 4):
        z_u, z_i, f_u, f_i = layer(
            iu2, ii2, z_u, z_i, dis_u, dis_i, f_u, f_i, lw_splats[l])
    return _mk_score()(user_ids, item_ids, f_u, f_i)


# speedup vs baseline: 23.8573x; 1.1759x over previous
import jax
import jax.numpy as jnp
from jax import lax
from jax.experimental import pallas as pl
from jax.experimental.pallas import tpu as pltpu
from jax.experimental.pallas import tpu_sc as plsc

NU = 25000
D = 64
B = 16384

NC, NS, L = 2, 16, 16
NPAD = 25088
RPT = NPAD // NS
RCH = 112
NRCH = RPT // RCH
ZROWS = NPAD + 8
EPAD = 401408
EPT = EPAD // NS
ECH = 128
NECH = EPT // ECH
BPT = B // (NC * NS)
BCH = 128

ECH2 = 64
NB = 4
QE = NB * ECH2
NQ = EPT // QE
RCHL = 28
NRCHL = RPT // RCHL

_MESH = plsc.VectorSubcoreMesh(
    core_axis_name="c", subcore_axis_name="s", num_cores=NC, num_subcores=NS)

_f32 = jnp.float32
_i32 = jnp.int32


def _fori(n, body):
    lax.fori_loop(0, n, lambda i, c: (body(i), c)[1], 0)


def _fill_zero_2d(ref, rows):
    zero = jnp.zeros((L,), _f32)

    def row(r):
        for j in range(D // L):
            ref[r, pl.ds(j * L, L)] = zero
    _fori(rows, row)


def _rsqrt16(x):
    i = lax.bitcast_convert_type(x, _i32)
    y = lax.bitcast_convert_type(jnp.int32(0x5F3759DF) - (i >> 1), _f32)
    for _ in range(3):
        y = y * (1.5 - 0.5 * x * y * y)
    return jnp.where(x >= 0.5, y, 0.0)


def _prologue_body(iu, ii, ue, ie, lw0v,
                   z_u, z_i, f_u, f_i, dis_u, dis_i,
                   deg_sh, idxa, idxb, ones_v, col_v,
                   row_v0, row_v1, zv0, zv1, lw_v,
                   ss0, ss1, ss2, ss3, sia, sib):
    c = lax.axis_index("c")
    s = lax.axis_index("s")
    base = s * RPT
    idx = [idxa, idxb]
    sem_i = [sia, sib]
    sem_s = [ss0, ss1, ss2, ss3]
    row_v = [row_v0, row_v1]
    zv = [zv0, zv1]

    def ones_row(r):
        ones_v[r] = jnp.ones((L,), _f32)
    _fori(ECH2, ones_row)

    def zcol(r):
        col_v[r] = jnp.zeros((L,), _f32)
    _fori(RPT, zcol)
    pltpu.sync_copy(lw0v, lw_v)

    def half(inter_ref, emb_ref, z_out, f_out, dis_out):
        pltpu.sync_copy(col_v, deg_sh.at[pl.ds(base, RPT)])

        @pl.when(s == NS - 1)
        def _():
            pltpu.sync_copy(col_v.at[pl.ds(0, 8)], deg_sh.at[pl.ds(NPAD, 8)])
        plsc.subcore_barrier()

        qrow = s * (EPT // ECH2)

        def idx_issue(q, p):
            pltpu.async_copy(inter_ref.at[pl.ds(qrow + q * NB, NB)],
                             idx[p], sem_i[p])

        def scat_wait(p):
            for k in range(NB):
                pltpu.make_async_copy(
                    ones_v, deg_sh.at[idx[p].at[k]], sem_s[k]).wait()

        def cquad(q, p, wait_prev, prefetch):
            if wait_prev:
                scat_wait(1 - p)
            if prefetch is None:
                idx_issue(q + 1, 1 - p)
            else:
                pl.when(prefetch)(lambda: idx_issue(q + 1, 1 - p))
            pltpu.make_async_copy(inter_ref.at[pl.ds(0, NB)], idx[p],
                                  sem_i[p]).wait()
            for k in range(NB):
                pltpu.async_copy(ones_v, deg_sh.at[idx[p].at[k]], sem_s[k],
                                 add=True)

        idx_issue(0, 0)
        cquad(0, 0, False, None)
        cquad(1, 1, True, None)

        def qpair(t):
            cquad(2 * t, 0, True, None)
            cquad(2 * t + 1, 1, True, t < NQ // 2 - 1)
        lax.fori_loop(1, NQ // 2, lambda t, cc: (qpair(t), cc)[1], 0)
        scat_wait(1)
        plsc.subcore_barrier()

        pltpu.sync_copy(deg_sh.at[pl.ds(base, RPT)], col_v)

        def drow(r):
            col_v[r] = _rsqrt16(col_v[r])
        _fori(RPT, drow)
        pltpu.sync_copy(col_v, dis_out.at[pl.ds(base, RPT)])

        lw0 = lw_v[...]

        def e_load(rc, p):
            pltpu.async_copy(emb_ref.at[pl.ds(base + rc * RCH, RCH)],
                             row_v[p], sem_i[p])

        def e_load_wait(p):
            pltpu.make_async_copy(emb_ref.at[pl.ds(base, RCH)], row_v[p],
                                  sem_i[p]).wait()

        def e_stores_wait(p):
            pltpu.make_async_copy(zv[p], z_out.at[pl.ds(base, RCH)],
                                  sem_s[p]).wait()
            pltpu.make_async_copy(row_v[p], f_out.at[pl.ds(base, RCH)],
                                  sem_s[p]).wait()

        def e_chunk(rc, p, mode):
            if mode == "first":
                e_load(1, 1)
            elif mode == "mid":
                e_stores_wait(1 - p)
                e_load(rc + 1, 1 - p)
            e_load_wait(p)
            rbase = base + rc * RCH

            def row(r):
                d = col_v[rc * RCH + r]
                for j in range(D // L):
                    sl = pl.ds(j * L, L)
                    x = row_v[p][r, sl]
                    zv[p][r, sl] = d * x
                    row_v[p][r, sl] = lw0 * x
            _fori(RCH, row)
            pltpu.async_copy(zv[p], z_out.at[pl.ds(rbase, RCH)], sem_s[p])
            pltpu.async_copy(row_v[p], f_out.at[pl.ds(rbase, RCH)],
                             sem_s[p])

        e_load(0, 0)
        e_chunk(0, 0, "first")

        def epair(t):
            e_chunk(2 * t + 1, 1, "mid")
            e_chunk(2 * t + 2, 0, "mid")
        lax.fori_loop(0, NRCH // 2 - 1, lambda t, cc: (epair(t), cc)[1], 0)
        e_chunk(NRCH - 1, 1, "last")
        e_stores_wait(0)
        e_stores_wait(1)

        @pl.when(s == NS - 1)
        def _():
            _fill_zero_2d(zv0, 8)
            pltpu.sync_copy(zv0.at[pl.ds(0, 8)], z_out.at[pl.ds(NPAD, 8)])

    @pl.when(c == 0)
    def _():
        half(iu, ue, z_u, f_u, dis_u)

    @pl.when(c == 1)
    def _():
        half(ii, ie, z_i, f_i, dis_i)


def _layer_body(iu, ii, z_u, z_i, dis_u, dis_i, f_u, f_i, lwlv,
                z_u2, z_i2, f_u2, f_i2,
                acc_sh, idxg_a, idxg_b, idxs_a, idxs_b,
                rows0, rows1, rows2, rows3,
                acc_v0, acc_v1, fin_v0, fin_v1, dis_c0, dis_c1, lw_v,
                sg0, sg1, sg2, sg3, ss0, ss1, ss2, ss3, sia, sib):
    c = lax.axis_index("c")
    s = lax.axis_index("s")
    base = s * RPT
    rows = [rows0, rows1, rows2, rows3]
    sem_g = [sg0, sg1, sg2, sg3]
    sem_s = [ss0, ss1, ss2, ss3]
    idx_g = [idxg_a, idxg_b]
    idx_s = [idxs_a, idxs_b]
    sem_i = [sia, sib]
    acc_v = [acc_v0, acc_v1]
    fin_v = [fin_v0, fin_v1]
    dis_c = [dis_c0, dis_c1]

    pltpu.sync_copy(lwlv, lw_v)

    def half(gidx_ref, sidx_ref, z_tab, dis_ref, f_in, z_out, f_out):
        _fill_zero_2d(fin_v0, RCHL)

        def zgrp(g):
            for k in range(8):
                pltpu.async_copy(
                    fin_v0,
                    acc_sh.at[pl.ds(base + (g * 8 + k) * RCHL, RCHL)], ss0)
            for k in range(8):
                pltpu.make_async_copy(
                    fin_v0,
                    acc_sh.at[pl.ds(base + (g * 8 + k) * RCHL, RCHL)],
                    ss0).wait()
        _fori(NRCHL // 8, zgrp)

        @pl.when(s == NS - 1)
        def _():
            pltpu.sync_copy(fin_v0.at[pl.ds(0, 8)], acc_sh.at[pl.ds(NPAD, 8)])
        plsc.subcore_barrier()

        qrow = s * (EPT // ECH2)

        def idx_issue(q, p):
            r0 = qrow + q * NB
            pltpu.async_copy(gidx_ref.at[pl.ds(r0, NB)], idx_g[p], sem_i[p])
            pltpu.async_copy(sidx_ref.at[pl.ds(r0, NB)], idx_s[p], sem_i[p])

        def idx_wait(p):
            pltpu.make_async_copy(
                gidx_ref.at[pl.ds(0, NB)], idx_g[p], sem_i[p]).wait()
            pltpu.make_async_copy(
                gidx_ref.at[pl.ds(0, NB)], idx_s[p], sem_i[p]).wait()

        def scat_wait(p):
            for k in range(NB):
                pltpu.make_async_copy(
                    rows[k], acc_sh.at[idx_s[p].at[k]], sem_s[k]).wait()

        def quad_step(q, p, wait_prev, prefetch):
            if wait_prev:
                scat_wait(1 - p)
            if prefetch is None:
                idx_issue(q + 1, 1 - p)
            else:
                pl.when(prefetch)(lambda: idx_issue(q + 1, 1 - p))
            idx_wait(p)
            for k in range(NB):
                pltpu.async_copy(z_tab.at[idx_g[p].at[k]], rows[k], sem_g[k])
            for k in range(NB):
                pltpu.make_async_copy(
                    z_tab.at[idx_g[p].at[k]], rows[k], sem_g[k]).wait()
                pltpu.async_copy(
                    rows[k], acc_sh.at[idx_s[p].at[k]], sem_s[k], add=True)

        idx_issue(0, 0)
        quad_step(0, 0, False, None)
        quad_step(1, 1, True, None)

        def qpair(t):
            quad_step(2 * t, 0, True, None)
            quad_step(2 * t + 1, 1, True, t < NQ // 2 - 1)
        lax.fori_loop(1, NQ // 2, lambda t, cc: (qpair(t), cc)[1], 0)
        scat_wait(1)
        plsc.subcore_barrier()

        lwl = lw_v[...]

        def d_loads(rc, p):
            rbase = base + rc * RCHL
            pltpu.async_copy(f_in.at[pl.ds(rbase, RCHL)], fin_v[p],
                             sem_i[p])
            pltpu.async_copy(dis_ref.at[pl.ds(rbase, RCHL)], dis_c[p],
                             sem_i[p])

        def d_loads_wait(p):
            pltpu.make_async_copy(f_in.at[pl.ds(base, RCHL)], fin_v[p],
                                  sem_i[p]).wait()
            pltpu.make_async_copy(dis_ref.at[pl.ds(base, RCHL)], dis_c[p],
                                  sem_i[p]).wait()

        def d_stores_wait(p):
            pltpu.make_async_copy(acc_v[p], z_out.at[pl.ds(base, RCHL)],
                                  sem_s[p]).wait()
            pltpu.make_async_copy(fin_v[p], f_out.at[pl.ds(base, RCHL)],
                                  sem_s[p]).wait()

        def d_chunk(rc, p, mode):
            if mode == "first":
                d_loads(1, 1)
            elif mode == "mid":
                d_stores_wait(1 - p)
                d_loads(rc + 1, 1 - p)
            rbase = base + rc * RCHL
            pltpu.sync_copy(acc_sh.at[pl.ds(rbase, RCHL)], acc_v[p])
            d_loads_wait(p)

            def row(r):
                d = dis_c[p][r]
                for j in range(D // L):
                    sl = pl.ds(j * L, L)
                    t = d * acc_v[p][r, sl]
                    fin_v[p][r, sl] = fin_v[p][r, sl] + lwl * t
                    acc_v[p][r, sl] = d * t
            _fori(RCHL, row)
            pltpu.async_copy(acc_v[p], z_out.at[pl.ds(rbase, RCHL)],
                             sem_s[p])
            pltpu.async_copy(fin_v[p], f_out.at[pl.ds(rbase, RCHL)],
                             sem_s[p])

        d_loads(0, 0)
        d_chunk(0, 0, "first")

        def dpair(t):
            d_chunk(2 * t + 1, 1, "mid")
            d_chunk(2 * t + 2, 0, "mid")
        lax.fori_loop(0, NRCHL // 2 - 1,
                      lambda t, cc: (dpair(t), cc)[1], 0)
        d_chunk(NRCHL - 1, 1, "last")
        d_stores_wait(0)
        d_stores_wait(1)

        @pl.when(s == NS - 1)
        def _():
            _fill_zero_2d(fin_v0, 8)
            pltpu.sync_copy(fin_v0.at[pl.ds(0, 8)], z_out.at[pl.ds(NPAD, 8)])

    @pl.when(c == 0)
    def _():
        half(ii, iu, z_i, dis_u, f_u, z_u2, f_u2)

    @pl.when(c == 1)
    def _():
        half(iu, ii, z_u, dis_i, f_i, z_i2, f_i2)


def _score_body(uid, iid, f_u, f_i, out,
                uid_v, iid_v, urows, irows, sc_v):
    c = lax.axis_index("c")
    s = lax.axis_index("s")
    wid = s * NC + c
    tb = wid * BPT

    def bchunk(bc):
        off = tb + bc * BCH
        pltpu.sync_copy(uid.at[pl.ds(off, BCH)], uid_v)
        pltpu.sync_copy(iid.at[pl.ds(off, BCH)], iid_v)
        pltpu.sync_copy(f_u.at[uid_v], urows)
        pltpu.sync_copy(f_i.at[iid_v], irows)

        lanes = lax.iota(_i32, L)

        def grp(g):
            def rb(k, acc):
                r = g * L + k
                p = jnp.zeros((L,), _f32)
                for j in range(D // L):
                    sl = pl.ds(j * L, L)
                    p = p + urows[r, sl] * irows[r, sl]
                for sh in (1, 2, 4, 8):
                    p = p + jnp.take(p, lanes ^ sh)
                return jnp.where(lanes == k, p, acc)
            acc = lax.fori_loop(0, L, rb, jnp.zeros((L,), _f32))
            sc_v[pl.ds(g * L, L)] = acc
        _fori(BCH // L, grp)
        pltpu.sync_copy(sc_v, out.at[pl.ds(off, BCH)])
    _fori(BPT // BCH, bchunk)


def _node_struct():
    return jax.ShapeDtypeStruct((NPAD, D), _f32)


def _mk_prologue():
    return pl.kernel(
        _prologue_body,
        out_type=[
            jax.ShapeDtypeStruct((ZROWS, D), _f32),
            jax.ShapeDtypeStruct((ZROWS, D), _f32),
            _node_struct(),
            _node_struct(),
            jax.ShapeDtypeStruct((NPAD, L), _f32),
            jax.ShapeDtypeStruct((NPAD, L), _f32),
        ],
        mesh=_MESH,
        compiler_params=pltpu.CompilerParams(use_tc_tiling_on_sc=False),
        scratch_types=[
            pltpu.VMEM_SHARED((ZROWS, L), _f32),
            pltpu.VMEM((NB, ECH2), _i32),
            pltpu.VMEM((NB, ECH2), _i32),
            pltpu.VMEM((ECH2, L), _f32),
            pltpu.VMEM((RPT, L), _f32),
            pltpu.VMEM((RCH, D), _f32),
            pltpu.VMEM((RCH, D), _f32),
            pltpu.VMEM((RCH, D), _f32),
            pltpu.VMEM((RCH, D), _f32),
            pltpu.VMEM((L,), _f32),
            pltpu.SemaphoreType.DMA,
            pltpu.SemaphoreType.DMA,
            pltpu.SemaphoreType.DMA,
            pltpu.SemaphoreType.DMA,
            pltpu.SemaphoreType.DMA,
            pltpu.SemaphoreType.DMA,
        ],
    )


def _mk_layer():
    return pl.kernel(
        _layer_body,
        out_type=[
            jax.ShapeDtypeStruct((ZROWS, D), _f32),
            jax.ShapeDtypeStruct((ZROWS, D), _f32),
            _node_struct(),
            _node_struct(),
        ],
        mesh=_MESH,
        compiler_params=pltpu.CompilerParams(use_tc_tiling_on_sc=False),
        scratch_types=[
            pltpu.VMEM_SHARED((ZROWS, D), _f32),
            pltpu.VMEM((NB, ECH2), _i32),
            pltpu.VMEM((NB, ECH2), _i32),
            pltpu.VMEM((NB, ECH2), _i32),
            pltpu.VMEM((NB, ECH2), _i32),
            pltpu.VMEM((ECH2, D), _f32),
            pltpu.VMEM((ECH2, D), _f32),
            pltpu.VMEM((ECH2, D), _f32),
            pltpu.VMEM((ECH2, D), _f32),
            pltpu.VMEM((RCHL, D), _f32),
            pltpu.VMEM((RCHL, D), _f32),
            pltpu.VMEM((RCHL, D), _f32),
            pltpu.VMEM((RCHL, D), _f32),
            pltpu.VMEM((RCHL, L), _f32),
            pltpu.VMEM((RCHL, L), _f32),
            pltpu.VMEM((L,), _f32),
            pltpu.SemaphoreType.DMA,
            pltpu.SemaphoreType.DMA,
            pltpu.SemaphoreType.DMA,
            pltpu.SemaphoreType.DMA,
            pltpu.SemaphoreType.DMA,
            pltpu.SemaphoreType.DMA,
            pltpu.SemaphoreType.DMA,
            pltpu.SemaphoreType.DMA,
            pltpu.SemaphoreType.DMA,
            pltpu.SemaphoreType.DMA,
        ],
    )


def _mk_score():
    return pl.kernel(
        _score_body,
        out_type=jax.ShapeDtypeStruct((B,), _f32),
        mesh=_MESH,
        compiler_params=pltpu.CompilerParams(use_tc_tiling_on_sc=False),
        scratch_types=[
            pltpu.VMEM((BCH,), _i32),
            pltpu.VMEM((BCH,), _i32),
            pltpu.VMEM((BCH, D), _f32),
            pltpu.VMEM((BCH, D), _f32),
            pltpu.VMEM((BCH,), _f32),
        ],
    )


def kernel(user_ids, item_ids, inter_u, inter_i, user_emb, item_emb,
           layer_weights):
    lw = jax.nn.softmax(layer_weights)
    lw_splats = [jnp.full((L,), lw[k], _f32) for k in range(4)]
    ue_p = jnp.pad(user_emb, ((0, NPAD - NU), (0, 0)))
    ie_p = jnp.pad(item_emb, ((0, NPAD - NU), (0, 0)))
    iu_p = jnp.pad(inter_u, (0, EPAD - inter_u.shape[0]), constant_values=NPAD)
    ii_p = jnp.pad(inter_i, (0, EPAD - inter_i.shape[0]), constant_values=NPAD)

    iu2 = iu_p.reshape(EPAD // ECH2, ECH2)
    ii2 = ii_p.reshape(EPAD // ECH2, ECH2)

    z_u, z_i, f_u, f_i, dis_u, dis_i = _mk_prologue()(
        iu2, ii2, ue_p, ie_p, lw_splats[0])
    layer = _mk_layer()
    for l in range(1, 4):
        z_u, z_i, f_u, f_i = layer(
            iu2, ii2, z_u, z_i, dis_u, dis_i, f_u, f_i, lw_splats[l])
    return _mk_score()(user_ids, item_ids, f_u, f_i)


# speedup vs baseline: 24.1187x; 1.0110x over previous
import jax
import jax.numpy as jnp
from jax import lax
from jax.experimental import pallas as pl
from jax.experimental.pallas import tpu as pltpu
from jax.experimental.pallas import tpu_sc as plsc

NU = 25000
D = 64
B = 16384

NC, NS, L = 2, 16, 16
NPAD = 25088
RPT = NPAD // NS
RCH = 112
NRCH = RPT // RCH
ZROWS = NPAD + 8
EPAD = 401408
EPT = EPAD // NS
ECH = 128
NECH = EPT // ECH
BPT = B // (NC * NS)
BCH = 128

ECH2 = 64
NB = 4
QE = NB * ECH2
NQ = EPT // QE
RCHL = 28
NRCHL = RPT // RCHL
CCH = 112
NCQ = EPT // (NB * CCH)

_MESH = plsc.VectorSubcoreMesh(
    core_axis_name="c", subcore_axis_name="s", num_cores=NC, num_subcores=NS)

_f32 = jnp.float32
_i32 = jnp.int32


def _fori(n, body):
    lax.fori_loop(0, n, lambda i, c: (body(i), c)[1], 0)


def _fill_zero_2d(ref, rows):
    zero = jnp.zeros((L,), _f32)

    def row(r):
        for j in range(D // L):
            ref[r, pl.ds(j * L, L)] = zero
    _fori(rows, row)


def _rsqrt16(x):
    i = lax.bitcast_convert_type(x, _i32)
    y = lax.bitcast_convert_type(jnp.int32(0x5F3759DF) - (i >> 1), _f32)
    for _ in range(3):
        y = y * (1.5 - 0.5 * x * y * y)
    return jnp.where(x >= 0.5, y, 0.0)


def _prologue_body(iu, ii, ue, ie, lw0v,
                   z_u, z_i, f_u, f_i, dis_u, dis_i,
                   deg_sh, idxa, idxb, ones_v, col_v,
                   row_v0, row_v1, zv0, zv1, lw_v,
                   ss0, ss1, ss2, ss3, sia, sib):
    c = lax.axis_index("c")
    s = lax.axis_index("s")
    base = s * RPT
    idx = [idxa, idxb]
    sem_i = [sia, sib]
    sem_s = [ss0, ss1, ss2, ss3]
    row_v = [row_v0, row_v1]
    zv = [zv0, zv1]

    def ones_row(r):
        ones_v[r] = jnp.ones((L,), _f32)
    _fori(CCH, ones_row)

    def zcol(r):
        col_v[r] = jnp.zeros((L,), _f32)
    _fori(RPT, zcol)
    pltpu.sync_copy(lw0v, lw_v)

    def half(inter_ref, emb_ref, z_out, f_out, dis_out):
        pltpu.sync_copy(col_v, deg_sh.at[pl.ds(base, RPT)])

        @pl.when(s == NS - 1)
        def _():
            pltpu.sync_copy(col_v.at[pl.ds(0, 8)], deg_sh.at[pl.ds(NPAD, 8)])
        plsc.subcore_barrier()

        qrow = s * (EPT // CCH)

        def idx_issue(q, p):
            pltpu.async_copy(inter_ref.at[pl.ds(qrow + q * NB, NB)],
                             idx[p], sem_i[p])

        def scat_wait(p):
            for k in range(NB):
                pltpu.make_async_copy(
                    ones_v, deg_sh.at[idx[p].at[k]], sem_s[k]).wait()

        def cquad(q, p, wait_prev, prefetch):
            if wait_prev:
                scat_wait(1 - p)
            if prefetch is None:
                idx_issue(q + 1, 1 - p)
            else:
                pl.when(prefetch)(lambda: idx_issue(q + 1, 1 - p))
            pltpu.make_async_copy(inter_ref.at[pl.ds(0, NB)], idx[p],
                                  sem_i[p]).wait()
            for k in range(NB):
                pltpu.async_copy(ones_v, deg_sh.at[idx[p].at[k]], sem_s[k],
                                 add=True)

        idx_issue(0, 0)
        cquad(0, 0, False, None)
        cquad(1, 1, True, None)

        def qpair(t):
            cquad(2 * t, 0, True, None)
            cquad(2 * t + 1, 1, True, t < NCQ // 2 - 1)
        lax.fori_loop(1, NCQ // 2, lambda t, cc: (qpair(t), cc)[1], 0)
        scat_wait(1)
        plsc.subcore_barrier()

        pltpu.sync_copy(deg_sh.at[pl.ds(base, RPT)], col_v)

        def drow(r):
            col_v[r] = _rsqrt16(col_v[r])
        _fori(RPT, drow)
        pltpu.sync_copy(col_v, dis_out.at[pl.ds(base, RPT)])

        lw0 = lw_v[...]

        def e_load(rc, p):
            pltpu.async_copy(emb_ref.at[pl.ds(base + rc * RCH, RCH)],
                             row_v[p], sem_i[p])

        def e_load_wait(p):
            pltpu.make_async_copy(emb_ref.at[pl.ds(base, RCH)], row_v[p],
                                  sem_i[p]).wait()

        def e_stores_wait(p):
            pltpu.make_async_copy(zv[p], z_out.at[pl.ds(base, RCH)],
                                  sem_s[p]).wait()
            pltpu.make_async_copy(row_v[p], f_out.at[pl.ds(base, RCH)],
                                  sem_s[p]).wait()

        def e_chunk(rc, p, mode):
            if mode == "first":
                e_load(1, 1)
            elif mode == "mid":
                e_stores_wait(1 - p)
                e_load(rc + 1, 1 - p)
            e_load_wait(p)
            rbase = base + rc * RCH

            def row(r):
                d = col_v[rc * RCH + r]
                for j in range(D // L):
                    sl = pl.ds(j * L, L)
                    x = row_v[p][r, sl]
                    zv[p][r, sl] = d * x
                    row_v[p][r, sl] = lw0 * x
            _fori(RCH, row)
            pltpu.async_copy(zv[p], z_out.at[pl.ds(rbase, RCH)], sem_s[p])
            pltpu.async_copy(row_v[p], f_out.at[pl.ds(rbase, RCH)],
                             sem_s[p])

        e_load(0, 0)
        e_chunk(0, 0, "first")

        def epair(t):
            e_chunk(2 * t + 1, 1, "mid")
            e_chunk(2 * t + 2, 0, "mid")
        lax.fori_loop(0, NRCH // 2 - 1, lambda t, cc: (epair(t), cc)[1], 0)
        e_chunk(NRCH - 1, 1, "last")
        e_stores_wait(0)
        e_stores_wait(1)

        @pl.when(s == NS - 1)
        def _():
            _fill_zero_2d(zv0, 8)
            pltpu.sync_copy(zv0.at[pl.ds(0, 8)], z_out.at[pl.ds(NPAD, 8)])

    @pl.when(c == 0)
    def _():
        half(iu, ue, z_u, f_u, dis_u)

    @pl.when(c == 1)
    def _():
        half(ii, ie, z_i, f_i, dis_i)


def _layer_body(iu, ii, z_u, z_i, dis_u, dis_i, f_u, f_i, lwlv,
                z_u2, z_i2, f_u2, f_i2,
                acc_sh, idxg_a, idxg_b, idxs_a, idxs_b,
                rows0, rows1, rows2, rows3,
                acc_v0, acc_v1, fin_v0, fin_v1, dis_c0, dis_c1, lw_v,
                sg0, sg1, sg2, sg3, ss0, ss1, ss2, ss3, sia, sib):
    c = lax.axis_index("c")
    s = lax.axis_index("s")
    base = s * RPT
    rows = [rows0, rows1, rows2, rows3]
    sem_g = [sg0, sg1, sg2, sg3]
    sem_s = [ss0, ss1, ss2, ss3]
    idx_g = [idxg_a, idxg_b]
    idx_s = [idxs_a, idxs_b]
    sem_i = [sia, sib]
    acc_v = [acc_v0, acc_v1]
    fin_v = [fin_v0, fin_v1]
    dis_c = [dis_c0, dis_c1]

    pltpu.sync_copy(lwlv, lw_v)

    def half(gidx_ref, sidx_ref, z_tab, dis_ref, f_in, z_out, f_out):
        _fill_zero_2d(fin_v0, RCHL)

        def zgrp(g):
            for k in range(8):
                pltpu.async_copy(
                    fin_v0,
                    acc_sh.at[pl.ds(base + (g * 8 + k) * RCHL, RCHL)], ss0)
            for k in range(8):
                pltpu.make_async_copy(
                    fin_v0,
                    acc_sh.at[pl.ds(base + (g * 8 + k) * RCHL, RCHL)],
                    ss0).wait()
        _fori(NRCHL // 8, zgrp)

        @pl.when(s == NS - 1)
        def _():
            pltpu.sync_copy(fin_v0.at[pl.ds(0, 8)], acc_sh.at[pl.ds(NPAD, 8)])
        plsc.subcore_barrier()

        qrow = s * (EPT // ECH2)

        def idx_issue(q, p):
            r0 = qrow + q * NB
            pltpu.async_copy(gidx_ref.at[pl.ds(r0, NB)], idx_g[p], sem_i[p])
            pltpu.async_copy(sidx_ref.at[pl.ds(r0, NB)], idx_s[p], sem_i[p])

        def idx_wait(p):
            pltpu.make_async_copy(
                gidx_ref.at[pl.ds(0, NB)], idx_g[p], sem_i[p]).wait()
            pltpu.make_async_copy(
                gidx_ref.at[pl.ds(0, NB)], idx_s[p], sem_i[p]).wait()

        def scat_wait(p):
            for k in range(NB):
                pltpu.make_async_copy(
                    rows[k], acc_sh.at[idx_s[p].at[k]], sem_s[k]).wait()

        def quad_step(q, p, wait_prev, prefetch):
            if wait_prev:
                scat_wait(1 - p)
            if prefetch is None:
                idx_issue(q + 1, 1 - p)
            else:
                pl.when(prefetch)(lambda: idx_issue(q + 1, 1 - p))
            idx_wait(p)
            for k in range(NB):
                pltpu.async_copy(z_tab.at[idx_g[p].at[k]], rows[k], sem_g[k])
            for k in range(NB):
                pltpu.make_async_copy(
                    z_tab.at[idx_g[p].at[k]], rows[k], sem_g[k]).wait()
                pltpu.async_copy(
                    rows[k], acc_sh.at[idx_s[p].at[k]], sem_s[k], add=True)

        idx_issue(0, 0)
        quad_step(0, 0, False, None)
        quad_step(1, 1, True, None)

        def qpair(t):
            quad_step(2 * t, 0, True, None)
            quad_step(2 * t + 1, 1, True, t < NQ // 2 - 1)
        lax.fori_loop(1, NQ // 2, lambda t, cc: (qpair(t), cc)[1], 0)
        scat_wait(1)
        plsc.subcore_barrier()

        lwl = lw_v[...]

        def d_loads(rc, p):
            rbase = base + rc * RCHL
            pltpu.async_copy(f_in.at[pl.ds(rbase, RCHL)], fin_v[p],
                             sem_i[p])
            pltpu.async_copy(dis_ref.at[pl.ds(rbase, RCHL)], dis_c[p],
                             sem_i[p])

        def d_loads_wait(p):
            pltpu.make_async_copy(f_in.at[pl.ds(base, RCHL)], fin_v[p],
                                  sem_i[p]).wait()
            pltpu.make_async_copy(dis_ref.at[pl.ds(base, RCHL)], dis_c[p],
                                  sem_i[p]).wait()

        def d_stores_wait(p):
            pltpu.make_async_copy(acc_v[p], z_out.at[pl.ds(base, RCHL)],
                                  sem_s[p]).wait()
            pltpu.make_async_copy(fin_v[p], f_out.at[pl.ds(base, RCHL)],
                                  sem_s[p]).wait()

        def d_chunk(rc, p, mode):
            if mode == "first":
                d_loads(1, 1)
            elif mode == "mid":
                d_stores_wait(1 - p)
                d_loads(rc + 1, 1 - p)
            rbase = base + rc * RCHL
            pltpu.sync_copy(acc_sh.at[pl.ds(rbase, RCHL)], acc_v[p])
            d_loads_wait(p)

            def row(r):
                d = dis_c[p][r]
                for j in range(D // L):
                    sl = pl.ds(j * L, L)
                    t = d * acc_v[p][r, sl]
                    fin_v[p][r, sl] = fin_v[p][r, sl] + lwl * t
                    acc_v[p][r, sl] = d * t
            _fori(RCHL, row)
            pltpu.async_copy(acc_v[p], z_out.at[pl.ds(rbase, RCHL)],
                             sem_s[p])
            pltpu.async_copy(fin_v[p], f_out.at[pl.ds(rbase, RCHL)],
                             sem_s[p])

        d_loads(0, 0)
        d_chunk(0, 0, "first")

        def dpair(t):
            d_chunk(2 * t + 1, 1, "mid")
            d_chunk(2 * t + 2, 0, "mid")
        lax.fori_loop(0, NRCHL // 2 - 1,
                      lambda t, cc: (dpair(t), cc)[1], 0)
        d_chunk(NRCHL - 1, 1, "last")
        d_stores_wait(0)
        d_stores_wait(1)

        @pl.when(s == NS - 1)
        def _():
            _fill_zero_2d(fin_v0, 8)
            pltpu.sync_copy(fin_v0.at[pl.ds(0, 8)], z_out.at[pl.ds(NPAD, 8)])

    @pl.when(c == 0)
    def _():
        half(ii, iu, z_i, dis_u, f_u, z_u2, f_u2)

    @pl.when(c == 1)
    def _():
        half(iu, ii, z_u, dis_i, f_i, z_i2, f_i2)


def _score_body(uid, iid, f_u, f_i, out,
                uid_v, iid_v, urows, irows, sc_v):
    c = lax.axis_index("c")
    s = lax.axis_index("s")
    wid = s * NC + c
    tb = wid * BPT

    def bchunk(bc):
        off = tb + bc * BCH
        pltpu.sync_copy(uid.at[pl.ds(off, BCH)], uid_v)
        pltpu.sync_copy(iid.at[pl.ds(off, BCH)], iid_v)
        pltpu.sync_copy(f_u.at[uid_v], urows)
        pltpu.sync_copy(f_i.at[iid_v], irows)

        lanes = lax.iota(_i32, L)

        def grp(g):
            def rb(k, acc):
                r = g * L + k
                p = jnp.zeros((L,), _f32)
                for j in range(D // L):
                    sl = pl.ds(j * L, L)
                    p = p + urows[r, sl] * irows[r, sl]
                for sh in (1, 2, 4, 8):
                    p = p + jnp.take(p, lanes ^ sh)
                return jnp.where(lanes == k, p, acc)
            acc = lax.fori_loop(0, L, rb, jnp.zeros((L,), _f32))
            sc_v[pl.ds(g * L, L)] = acc
        _fori(BCH // L, grp)
        pltpu.sync_copy(sc_v, out.at[pl.ds(off, BCH)])
    _fori(BPT // BCH, bchunk)


def _node_struct():
    return jax.ShapeDtypeStruct((NPAD, D), _f32)


def _mk_prologue():
    return pl.kernel(
        _prologue_body,
        out_type=[
            jax.ShapeDtypeStruct((ZROWS, D), _f32),
            jax.ShapeDtypeStruct((ZROWS, D), _f32),
            _node_struct(),
            _node_struct(),
            jax.ShapeDtypeStruct((NPAD, L), _f32),
            jax.ShapeDtypeStruct((NPAD, L), _f32),
        ],
        mesh=_MESH,
        compiler_params=pltpu.CompilerParams(use_tc_tiling_on_sc=False),
        scratch_types=[
            pltpu.VMEM_SHARED((ZROWS, L), _f32),
            pltpu.VMEM((NB, CCH), _i32),
            pltpu.VMEM((NB, CCH), _i32),
            pltpu.VMEM((CCH, L), _f32),
            pltpu.VMEM((RPT, L), _f32),
            pltpu.VMEM((RCH, D), _f32),
            pltpu.VMEM((RCH, D), _f32),
            pltpu.VMEM((RCH, D), _f32),
            pltpu.VMEM((RCH, D), _f32),
            pltpu.VMEM((L,), _f32),
            pltpu.SemaphoreType.DMA,
            pltpu.SemaphoreType.DMA,
            pltpu.SemaphoreType.DMA,
            pltpu.SemaphoreType.DMA,
            pltpu.SemaphoreType.DMA,
            pltpu.SemaphoreType.DMA,
        ],
    )


def _mk_layer():
    return pl.kernel(
        _layer_body,
        out_type=[
            jax.ShapeDtypeStruct((ZROWS, D), _f32),
            jax.ShapeDtypeStruct((ZROWS, D), _f32),
            _node_struct(),
            _node_struct(),
        ],
        mesh=_MESH,
        compiler_params=pltpu.CompilerParams(use_tc_tiling_on_sc=False),
        scratch_types=[
            pltpu.VMEM_SHARED((ZROWS, D), _f32),
            pltpu.VMEM((NB, ECH2), _i32),
            pltpu.VMEM((NB, ECH2), _i32),
            pltpu.VMEM((NB, ECH2), _i32),
            pltpu.VMEM((NB, ECH2), _i32),
            pltpu.VMEM((ECH2, D), _f32),
            pltpu.VMEM((ECH2, D), _f32),
            pltpu.VMEM((ECH2, D), _f32),
            pltpu.VMEM((ECH2, D), _f32),
            pltpu.VMEM((RCHL, D), _f32),
            pltpu.VMEM((RCHL, D), _f32),
            pltpu.VMEM((RCHL, D), _f32),
            pltpu.VMEM((RCHL, D), _f32),
            pltpu.VMEM((RCHL, L), _f32),
            pltpu.VMEM((RCHL, L), _f32),
            pltpu.VMEM((L,), _f32),
            pltpu.SemaphoreType.DMA,
            pltpu.SemaphoreType.DMA,
            pltpu.SemaphoreType.DMA,
            pltpu.SemaphoreType.DMA,
            pltpu.SemaphoreType.DMA,
            pltpu.SemaphoreType.DMA,
            pltpu.SemaphoreType.DMA,
            pltpu.SemaphoreType.DMA,
            pltpu.SemaphoreType.DMA,
            pltpu.SemaphoreType.DMA,
        ],
    )


def _mk_score():
    return pl.kernel(
        _score_body,
        out_type=jax.ShapeDtypeStruct((B,), _f32),
        mesh=_MESH,
        compiler_params=pltpu.CompilerParams(use_tc_tiling_on_sc=False),
        scratch_types=[
            pltpu.VMEM((BCH,), _i32),
            pltpu.VMEM((BCH,), _i32),
            pltpu.VMEM((BCH, D), _f32),
            pltpu.VMEM((BCH, D), _f32),
            pltpu.VMEM((BCH,), _f32),
        ],
    )


def kernel(user_ids, item_ids, inter_u, inter_i, user_emb, item_emb,
           layer_weights):
    lw = jax.nn.softmax(layer_weights)
    lw_splats = [jnp.full((L,), lw[k], _f32) for k in range(4)]
    ue_p = jnp.pad(user_emb, ((0, NPAD - NU), (0, 0)))
    ie_p = jnp.pad(item_emb, ((0, NPAD - NU), (0, 0)))
    iu_p = jnp.pad(inter_u, (0, EPAD - inter_u.shape[0]), constant_values=NPAD)
    ii_p = jnp.pad(inter_i, (0, EPAD - inter_i.shape[0]), constant_values=NPAD)

    iu2 = iu_p.reshape(EPAD // ECH2, ECH2)
    ii2 = ii_p.reshape(EPAD // ECH2, ECH2)
    iu3 = iu_p.reshape(EPAD // CCH, CCH)
    ii3 = ii_p.reshape(EPAD // CCH, CCH)

    z_u, z_i, f_u, f_i, dis_u, dis_i = _mk_prologue()(
        iu3, ii3, ue_p, ie_p, lw_splats[0])
    layer = _mk_layer()
    for l in range(1, 4):
        z_u, z_i, f_u, f_i = layer(
            iu2, ii2, z_u, z_i, dis_u, dis_i, f_u, f_i, lw_splats[l])
    return _mk_score()(user_ids, item_ids, f_u, f_i)


# speedup vs baseline: 25.5316x; 1.0586x over previous
import jax
import jax.numpy as jnp
from jax import lax
from jax.experimental import pallas as pl
from jax.experimental.pallas import tpu as pltpu
from jax.experimental.pallas import tpu_sc as plsc

NU = 25000
D = 64
B = 16384

NC, NS, L = 2, 16, 16
NPAD = 25088
RPT = NPAD // NS
RCH = 112
NRCH = RPT // RCH
ZROWS = NPAD + 8
EPAD = 401408
EPT = EPAD // NS
ECH = 128
NECH = EPT // ECH
BPT = B // (NC * NS)
BCH = 128

ECH2 = 64
NB = 4
QE = NB * ECH2
NQ = EPT // QE
RCHL = 28
NRCHL = RPT // RCHL
CCH = 112
NCQ = EPT // (NB * CCH)

_MESH = plsc.VectorSubcoreMesh(
    core_axis_name="c", subcore_axis_name="s", num_cores=NC, num_subcores=NS)

_f32 = jnp.float32
_i32 = jnp.int32


def _fori(n, body):
    lax.fori_loop(0, n, lambda i, c: (body(i), c)[1], 0)


def _fill_zero_2d(ref, rows):
    zero = jnp.zeros((L,), _f32)

    def row(r):
        for j in range(D // L):
            ref[r, pl.ds(j * L, L)] = zero
    _fori(rows, row)


def _rsqrt16(x):
    i = lax.bitcast_convert_type(x, _i32)
    y = lax.bitcast_convert_type(jnp.int32(0x5F3759DF) - (i >> 1), _f32)
    for _ in range(3):
        y = y * (1.5 - 0.5 * x * y * y)
    return jnp.where(x >= 0.5, y, 0.0)


def _prologue_body(iu, ii, ue, ie, lw0v,
                   z_u, z_i, f_u, f_i, dis_u, dis_i,
                   deg_sh, idxa, idxb, ones_v, col_v,
                   row_v0, row_v1, zv0, zv1, lw_v,
                   ss0, ss1, ss2, ss3, sia, sib):
    c = lax.axis_index("c")
    s = lax.axis_index("s")
    base = s * RPT
    idx = [idxa, idxb]
    sem_i = [sia, sib]
    sem_s = [ss0, ss1, ss2, ss3]
    row_v = [row_v0, row_v1]
    zv = [zv0, zv1]

    def ones_row(r):
        ones_v[r] = jnp.ones((L,), _f32)
    _fori(CCH, ones_row)

    def zcol(r):
        col_v[r] = jnp.zeros((L,), _f32)
    _fori(RPT, zcol)
    pltpu.sync_copy(lw0v, lw_v)

    def half(inter_ref, emb_ref, z_out, f_out, dis_out):
        pltpu.sync_copy(col_v, deg_sh.at[pl.ds(base, RPT)])

        @pl.when(s == NS - 1)
        def _():
            pltpu.sync_copy(col_v.at[pl.ds(0, 8)], deg_sh.at[pl.ds(NPAD, 8)])
        plsc.subcore_barrier()

        qrow = s * (EPT // CCH)

        def idx_issue(q, p):
            pltpu.async_copy(inter_ref.at[pl.ds(qrow + q * NB, NB)],
                             idx[p], sem_i[p])

        def scat_wait(p):
            for k in range(NB):
                pltpu.make_async_copy(
                    ones_v, deg_sh.at[idx[p].at[k]], sem_s[k]).wait()

        def cquad(q, p, wait_prev, prefetch):
            if wait_prev:
                scat_wait(1 - p)
            if prefetch is None:
                idx_issue(q + 1, 1 - p)
            else:
                pl.when(prefetch)(lambda: idx_issue(q + 1, 1 - p))
            pltpu.make_async_copy(inter_ref.at[pl.ds(0, NB)], idx[p],
                                  sem_i[p]).wait()
            for k in range(NB):
                pltpu.async_copy(ones_v, deg_sh.at[idx[p].at[k]], sem_s[k],
                                 add=True)

        idx_issue(0, 0)
        cquad(0, 0, False, None)
        cquad(1, 1, True, None)

        def qpair(t):
            cquad(2 * t, 0, True, None)
            cquad(2 * t + 1, 1, True, t < NCQ // 2 - 1)
        lax.fori_loop(1, NCQ // 2, lambda t, cc: (qpair(t), cc)[1], 0)
        scat_wait(1)
        plsc.subcore_barrier()

        pltpu.sync_copy(deg_sh.at[pl.ds(base, RPT)], col_v)

        def drow(r):
            col_v[r] = _rsqrt16(col_v[r])
        _fori(RPT, drow)
        pltpu.sync_copy(col_v, dis_out.at[pl.ds(base, RPT)])

        lw0 = lw_v[...]

        def e_load(rc, p):
            pltpu.async_copy(emb_ref.at[pl.ds(base + rc * RCH, RCH)],
                             row_v[p], sem_i[p])

        def e_load_wait(p):
            pltpu.make_async_copy(emb_ref.at[pl.ds(base, RCH)], row_v[p],
                                  sem_i[p]).wait()

        def e_stores_wait(p):
            pltpu.make_async_copy(zv[p], z_out.at[pl.ds(base, RCH)],
                                  sem_s[p]).wait()
            pltpu.make_async_copy(row_v[p], f_out.at[pl.ds(base, RCH)],
                                  sem_s[p]).wait()

        def e_chunk(rc, p, mode):
            if mode == "first":
                e_load(1, 1)
            elif mode == "mid":
                e_stores_wait(1 - p)
                e_load(rc + 1, 1 - p)
            e_load_wait(p)
            rbase = base + rc * RCH

            def row(r):
                d = col_v[rc * RCH + r]
                for j in range(D // L):
                    sl = pl.ds(j * L, L)
                    x = row_v[p][r, sl]
                    zv[p][r, sl] = d * x
                    row_v[p][r, sl] = lw0 * x
            _fori(RCH, row)
            pltpu.async_copy(zv[p], z_out.at[pl.ds(rbase, RCH)], sem_s[p])
            pltpu.async_copy(row_v[p], f_out.at[pl.ds(rbase, RCH)],
                             sem_s[p])

        e_load(0, 0)
        e_chunk(0, 0, "first")

        def epair(t):
            e_chunk(2 * t + 1, 1, "mid")
            e_chunk(2 * t + 2, 0, "mid")
        lax.fori_loop(0, NRCH // 2 - 1, lambda t, cc: (epair(t), cc)[1], 0)
        e_chunk(NRCH - 1, 1, "last")
        e_stores_wait(0)
        e_stores_wait(1)

        @pl.when(s == NS - 1)
        def _():
            _fill_zero_2d(zv0, 8)
            pltpu.sync_copy(zv0.at[pl.ds(0, 8)], z_out.at[pl.ds(NPAD, 8)])

    @pl.when(c == 0)
    def _():
        half(iu, ue, z_u, f_u, dis_u)

    @pl.when(c == 1)
    def _():
        half(ii, ie, z_i, f_i, dis_i)


def _layer_body(iu, ii, z_u, z_i, dis_u, dis_i, f_u, f_i, lwlv,
                z_u2, z_i2, f_u2, f_i2,
                acc_sh, idxg_a, idxg_b, idxs_a, idxs_b,
                rows0, rows1, rows2, rows3,
                acc_v0, acc_v1, fin_v0, fin_v1, dis_c0, dis_c1, lw_v,
                sg0, sg1, sg2, sg3, ss0, ss1, ss2, ss3, sia, sib):
    c = lax.axis_index("c")
    s = lax.axis_index("s")
    base = s * RPT
    rows = [rows0, rows1, rows2, rows3]
    sem_g = [sg0, sg1, sg2, sg3]
    sem_s = [ss0, ss1, ss2, ss3]
    idx_g = [idxg_a, idxg_b]
    idx_s = [idxs_a, idxs_b]
    sem_i = [sia, sib]
    acc_v = [acc_v0, acc_v1]
    fin_v = [fin_v0, fin_v1]
    dis_c = [dis_c0, dis_c1]

    pltpu.sync_copy(lwlv, lw_v)

    def half(gidx_ref, sidx_ref, z_tab, dis_ref, f_in, z_out, f_out):
        _fill_zero_2d(fin_v0, RCHL)

        def zgrp(g):
            for k in range(8):
                pltpu.async_copy(
                    fin_v0,
                    acc_sh.at[pl.ds(base + (g * 8 + k) * RCHL, RCHL)], ss0)
            for k in range(8):
                pltpu.make_async_copy(
                    fin_v0,
                    acc_sh.at[pl.ds(base + (g * 8 + k) * RCHL, RCHL)],
                    ss0).wait()
        _fori(NRCHL // 8, zgrp)

        @pl.when(s == NS - 1)
        def _():
            pltpu.sync_copy(fin_v0.at[pl.ds(0, 8)], acc_sh.at[pl.ds(NPAD, 8)])
        plsc.subcore_barrier()

        qrow = s * (EPT // ECH2)

        def idx_issue(q, p):
            r0 = qrow + q * NB
            pltpu.async_copy(gidx_ref.at[pl.ds(r0, NB)], idx_g[p], sem_i[p])
            pltpu.async_copy(sidx_ref.at[pl.ds(r0, NB)], idx_s[p], sem_i[p])

        def idx_wait(p):
            pltpu.make_async_copy(
                gidx_ref.at[pl.ds(0, NB)], idx_g[p], sem_i[p]).wait()
            pltpu.make_async_copy(
                gidx_ref.at[pl.ds(0, NB)], idx_s[p], sem_i[p]).wait()

        def scat_wait(p):
            for k in range(NB):
                pltpu.make_async_copy(
                    rows[k], acc_sh.at[idx_s[p].at[k]], sem_s[k]).wait()

        def g_issue(p, k):
            pltpu.async_copy(z_tab.at[idx_g[p].at[k]], rows[k], sem_g[k])

        def g_wait(p, k):
            pltpu.make_async_copy(
                z_tab.at[idx_g[p].at[k]], rows[k], sem_g[k]).wait()

        def s_issue(p, k):
            pltpu.async_copy(
                rows[k], acc_sh.at[idx_s[p].at[k]], sem_s[k], add=True)

        def s_wait(p, k):
            pltpu.make_async_copy(
                rows[k], acc_sh.at[idx_s[p].at[k]], sem_s[k]).wait()

        def chunk_step(q, p, j, wait_old, lag, prefetch):
            if wait_old:
                s_wait(1 - p, j)
            if j == 3 and prefetch is not False:
                if prefetch is True or prefetch is None:
                    idx_issue(q + 1, 1 - p)
                else:
                    pl.when(prefetch)(lambda: idx_issue(q + 1, 1 - p))
            if j == 0:
                idx_wait(p)
            g_issue(p, j)
            if lag is not None:
                lp, lk = lag
                g_wait(lp, lk)
                s_issue(lp, lk)

        idx_issue(0, 0)
        chunk_step(0, 0, 0, False, None, False)
        chunk_step(0, 0, 1, False, (0, 0), False)
        chunk_step(0, 0, 2, False, (0, 1), False)
        chunk_step(0, 0, 3, False, (0, 2), True)

        def qstep(q, p, pf):
            chunk_step(q, p, 0, True, (1 - p, 3), False)
            chunk_step(q, p, 1, True, (p, 0), False)
            chunk_step(q, p, 2, True, (p, 1), False)
            chunk_step(q, p, 3, True, (p, 2), pf)

        qstep(1, 1, True)

        def qpair(t):
            qstep(2 * t, 0, True)
            qstep(2 * t + 1, 1, t < NQ // 2 - 1)
        lax.fori_loop(1, NQ // 2, lambda t, cc: (qpair(t), cc)[1], 0)
        g_wait(1, 3)
        s_issue(1, 3)
        scat_wait(1)
        plsc.subcore_barrier()

        lwl = lw_v[...]

        def d_loads(rc, p):
            rbase = base + rc * RCHL
            pltpu.async_copy(f_in.at[pl.ds(rbase, RCHL)], fin_v[p],
                             sem_i[p])
            pltpu.async_copy(dis_ref.at[pl.ds(rbase, RCHL)], dis_c[p],
                             sem_i[p])

        def d_loads_wait(p):
            pltpu.make_async_copy(f_in.at[pl.ds(base, RCHL)], fin_v[p],
                                  sem_i[p]).wait()
            pltpu.make_async_copy(dis_ref.at[pl.ds(base, RCHL)], dis_c[p],
                                  sem_i[p]).wait()

        def d_stores_wait(p):
            pltpu.make_async_copy(acc_v[p], z_out.at[pl.ds(base, RCHL)],
                                  sem_s[p]).wait()
            pltpu.make_async_copy(fin_v[p], f_out.at[pl.ds(base, RCHL)],
                                  sem_s[p]).wait()

        def d_chunk(rc, p, mode):
            if mode == "first":
                d_loads(1, 1)
            elif mode == "mid":
                d_stores_wait(1 - p)
                d_loads(rc + 1, 1 - p)
            rbase = base + rc * RCHL
            pltpu.sync_copy(acc_sh.at[pl.ds(rbase, RCHL)], acc_v[p])
            d_loads_wait(p)

            def row(r):
                d = dis_c[p][r]
                for j in range(D // L):
                    sl = pl.ds(j * L, L)
                    t = d * acc_v[p][r, sl]
                    fin_v[p][r, sl] = fin_v[p][r, sl] + lwl * t
                    acc_v[p][r, sl] = d * t
            _fori(RCHL, row)
            pltpu.async_copy(acc_v[p], z_out.at[pl.ds(rbase, RCHL)],
                             sem_s[p])
            pltpu.async_copy(fin_v[p], f_out.at[pl.ds(rbase, RCHL)],
                             sem_s[p])

        d_loads(0, 0)
        d_chunk(0, 0, "first")

        def dpair(t):
            d_chunk(2 * t + 1, 1, "mid")
            d_chunk(2 * t + 2, 0, "mid")
        lax.fori_loop(0, NRCHL // 2 - 1,
                      lambda t, cc: (dpair(t), cc)[1], 0)
        d_chunk(NRCHL - 1, 1, "last")
        d_stores_wait(0)
        d_stores_wait(1)

        @pl.when(s == NS - 1)
        def _():
            _fill_zero_2d(fin_v0, 8)
            pltpu.sync_copy(fin_v0.at[pl.ds(0, 8)], z_out.at[pl.ds(NPAD, 8)])

    @pl.when(c == 0)
    def _():
        half(ii, iu, z_i, dis_u, f_u, z_u2, f_u2)

    @pl.when(c == 1)
    def _():
        half(iu, ii, z_u, dis_i, f_i, z_i2, f_i2)


def _score_body(uid, iid, f_u, f_i, out,
                uid_v, iid_v, urows, irows, sc_v):
    c = lax.axis_index("c")
    s = lax.axis_index("s")
    wid = s * NC + c
    tb = wid * BPT

    def bchunk(bc):
        off = tb + bc * BCH
        pltpu.sync_copy(uid.at[pl.ds(off, BCH)], uid_v)
        pltpu.sync_copy(iid.at[pl.ds(off, BCH)], iid_v)
        pltpu.sync_copy(f_u.at[uid_v], urows)
        pltpu.sync_copy(f_i.at[iid_v], irows)

        lanes = lax.iota(_i32, L)

        def grp(g):
            def rb(k, acc):
                r = g * L + k
                p = jnp.zeros((L,), _f32)
                for j in range(D // L):
                    sl = pl.ds(j * L, L)
                    p = p + urows[r, sl] * irows[r, sl]
                for sh in (1, 2, 4, 8):
                    p = p + jnp.take(p, lanes ^ sh)
                return jnp.where(lanes == k, p, acc)
            acc = lax.fori_loop(0, L, rb, jnp.zeros((L,), _f32))
            sc_v[pl.ds(g * L, L)] = acc
        _fori(BCH // L, grp)
        pltpu.sync_copy(sc_v, out.at[pl.ds(off, BCH)])
    _fori(BPT // BCH, bchunk)


def _node_struct():
    return jax.ShapeDtypeStruct((NPAD, D), _f32)


def _mk_prologue():
    return pl.kernel(
        _prologue_body,
        out_type=[
            jax.ShapeDtypeStruct((ZROWS, D), _f32),
            jax.ShapeDtypeStruct((ZROWS, D), _f32),
            _node_struct(),
            _node_struct(),
            jax.ShapeDtypeStruct((NPAD, L), _f32),
            jax.ShapeDtypeStruct((NPAD, L), _f32),
        ],
        mesh=_MESH,
        compiler_params=pltpu.CompilerParams(use_tc_tiling_on_sc=False),
        scratch_types=[
            pltpu.VMEM_SHARED((ZROWS, L), _f32),
            pltpu.VMEM((NB, CCH), _i32),
            pltpu.VMEM((NB, CCH), _i32),
            pltpu.VMEM((CCH, L), _f32),
            pltpu.VMEM((RPT, L), _f32),
            pltpu.VMEM((RCH, D), _f32),
            pltpu.VMEM((RCH, D), _f32),
            pltpu.VMEM((RCH, D), _f32),
            pltpu.VMEM((RCH, D), _f32),
            pltpu.VMEM((L,), _f32),
            pltpu.SemaphoreType.DMA,
            pltpu.SemaphoreType.DMA,
            pltpu.SemaphoreType.DMA,
            pltpu.SemaphoreType.DMA,
            pltpu.SemaphoreType.DMA,
            pltpu.SemaphoreType.DMA,
        ],
    )


def _mk_layer():
    return pl.kernel(
        _layer_body,
        out_type=[
            jax.ShapeDtypeStruct((ZROWS, D), _f32),
            jax.ShapeDtypeStruct((ZROWS, D), _f32),
            _node_struct(),
            _node_struct(),
        ],
        mesh=_MESH,
        compiler_params=pltpu.CompilerParams(use_tc_tiling_on_sc=False),
        scratch_types=[
            pltpu.VMEM_SHARED((ZROWS, D), _f32),
            pltpu.VMEM((NB, ECH2), _i32),
            pltpu.VMEM((NB, ECH2), _i32),
            pltpu.VMEM((NB, ECH2), _i32),
            pltpu.VMEM((NB, ECH2), _i32),
            pltpu.VMEM((ECH2, D), _f32),
            pltpu.VMEM((ECH2, D), _f32),
            pltpu.VMEM((ECH2, D), _f32),
            pltpu.VMEM((ECH2, D), _f32),
            pltpu.VMEM((RCHL, D), _f32),
            pltpu.VMEM((RCHL, D), _f32),
            pltpu.VMEM((RCHL, D), _f32),
            pltpu.VMEM((RCHL, D), _f32),
            pltpu.VMEM((RCHL, L), _f32),
            pltpu.VMEM((RCHL, L), _f32),
            pltpu.VMEM((L,), _f32),
            pltpu.SemaphoreType.DMA,
            pltpu.SemaphoreType.DMA,
            pltpu.SemaphoreType.DMA,
            pltpu.SemaphoreType.DMA,
            pltpu.SemaphoreType.DMA,
            pltpu.SemaphoreType.DMA,
            pltpu.SemaphoreType.DMA,
            pltpu.SemaphoreType.DMA,
            pltpu.SemaphoreType.DMA,
            pltpu.SemaphoreType.DMA,
        ],
    )


def _mk_score():
    return pl.kernel(
        _score_body,
        out_type=jax.ShapeDtypeStruct((B,), _f32),
        mesh=_MESH,
        compiler_params=pltpu.CompilerParams(use_tc_tiling_on_sc=False),
        scratch_types=[
            pltpu.VMEM((BCH,), _i32),
            pltpu.VMEM((BCH,), _i32),
            pltpu.VMEM((BCH, D), _f32),
            pltpu.VMEM((BCH, D), _f32),
            pltpu.VMEM((BCH,), _f32),
        ],
    )


def kernel(user_ids, item_ids, inter_u, inter_i, user_emb, item_emb,
           layer_weights):
    lw = jax.nn.softmax(layer_weights)
    lw_splats = [jnp.full((L,), lw[k], _f32) for k in range(4)]
    ue_p = jnp.pad(user_emb, ((0, NPAD - NU), (0, 0)))
    ie_p = jnp.pad(item_emb, ((0, NPAD - NU), (0, 0)))
    iu_p = jnp.pad(inter_u, (0, EPAD - inter_u.shape[0]), constant_values=NPAD)
    ii_p = jnp.pad(inter_i, (0, EPAD - inter_i.shape[0]), constant_values=NPAD)

    iu2 = iu_p.reshape(EPAD // ECH2, ECH2)
    ii2 = ii_p.reshape(EPAD // ECH2, ECH2)
    iu3 = iu_p.reshape(EPAD // CCH, CCH)
    ii3 = ii_p.reshape(EPAD // CCH, CCH)

    z_u, z_i, f_u, f_i, dis_u, dis_i = _mk_prologue()(
        iu3, ii3, ue_p, ie_p, lw_splats[0])
    layer = _mk_layer()
    for l in range(1, 4):
        z_u, z_i, f_u, f_i = layer(
            iu2, ii2, z_u, z_i, dis_u, dis_i, f_u, f_i, lw_splats[l])
    return _mk_score()(user_ids, item_ids, f_u, f_i)
